# baseline (speedup 1.0000x reference)
import functools

import jax
import jax.numpy as jnp
from jax import lax
from jax.experimental import pallas as pl
from jax.experimental.pallas import tpu as pltpu
from jax.experimental.pallas import tpu_sc as plsc

N_USERS = 10000
N_ITEMS = 10000
D = 128
E = 320000

NC = 2
NS = 16
L = 16
NW = NC * NS
N_TOTAL = 2 * E
EW = N_TOTAL // NW
CHUNK = 80
NCHUNK = EW // CHUNK
G = CHUNK // L
NB = 2


def _edge_scores(user_embedding, item_embedding, u_idx, v_idx):
  mesh = plsc.VectorSubcoreMesh(core_axis_name="c", subcore_axis_name="s")

  @functools.partial(
      pl.kernel,
      mesh=mesh,
      compiler_params=pltpu.CompilerParams(needs_layout_passes=False,
                                           use_tc_tiling_on_sc=False),
      out_type=jax.ShapeDtypeStruct((N_TOTAL,), jnp.float32),
      scratch_types=[
          pltpu.VMEM((EW,), jnp.int32),
          pltpu.VMEM((EW,), jnp.int32),
          pltpu.VMEM((NB, CHUNK, D // 2), jnp.float32),
          pltpu.VMEM((NB, CHUNK, D // 2), jnp.float32),
          pltpu.VMEM((NB, CHUNK), jnp.float32),
          pltpu.VMEM((L * L,), jnp.float32),
          pltpu.SemaphoreType.DMA,
          pltpu.SemaphoreType.DMA,
          pltpu.SemaphoreType.DMA,
          pltpu.SemaphoreType.DMA,
      ],
  )
  def k(uemb, iemb, uidx, vidx, out, uix_v, vix_v, urows, vrows, outv, mat,
        g0, g1, o0, o1):
    gsems = (g0, g1)
    osems = (o0, o1)
    wid = lax.axis_index("s") * NC + lax.axis_index("c")
    base = wid * EW
    lane16 = lax.iota(jnp.int32, L) * L

    pltpu.sync_copy(uidx.at[pl.ds(base, EW)], uix_v)
    pltpu.sync_copy(vidx.at[pl.ds(base, EW)], vix_v)

    def fire(cc, b):
      off = pl.multiple_of(cc * CHUNK, CHUNK)
      pltpu.async_copy(uemb.at[uix_v.at[pl.ds(off, CHUNK)]], urows.at[b],
                       gsems[b])
      pltpu.async_copy(iemb.at[vix_v.at[pl.ds(off, CHUNK)]], vrows.at[b],
                       gsems[b])

    def wait_gather(b):
      pltpu.make_async_copy(uemb.at[uix_v.at[pl.ds(0, CHUNK)]], urows.at[b],
                            gsems[b]).wait()
      pltpu.make_async_copy(iemb.at[vix_v.at[pl.ds(0, CHUNK)]], vrows.at[b],
                            gsems[b]).wait()

    def wait_out(b):
      pltpu.make_async_copy(outv.at[b], out.at[pl.ds(0, CHUNK)],
                            osems[b]).wait()

    fire(0, 0)

    def pair_body(i, _):
      for b in range(NB):
        cc = i * NB + b
        nb = (b + 1) % NB

        @pl.when(cc + 1 < NCHUNK)
        def _():
          fire(cc + 1, nb)

        @pl.when(cc >= NB)
        def _():
          wait_out(b)

        wait_gather(b)

        def group_body(g, _):
          accs = []
          for t in range(L):
            e = g * L + t
            acc0 = acc1 = None
            for j in range(D // (2 * L)):
              uw = plsc.bitcast(urows[b, e, pl.ds(j * L, L)], jnp.bfloat16)
              vw = plsc.bitcast(vrows[b, e, pl.ds(j * L, L)], jnp.bfloat16)
              pw = plsc.bitcast(uw * vw, jnp.uint32)
              hi = plsc.bitcast(pw & jnp.uint32(0xFFFF0000), jnp.float32)
              lo = plsc.bitcast(pw << jnp.uint32(16), jnp.float32)
              acc0 = hi if acc0 is None else acc0 + hi
              acc1 = lo if acc1 is None else acc1 + lo
            accs.append(acc0 + acc1)
          outv[b, pl.ds(g * L, L)] = jnp.zeros((L,), jnp.float32)
          for t in range(L):
            eidx = jnp.full((L,), g * L + t, jnp.int32)
            plsc.addupdate_scatter(outv.at[b], [eidx], accs[t])
          return 0

        lax.fori_loop(0, G, group_body, 0, unroll=True)
        obase = pl.multiple_of(base + cc * CHUNK, CHUNK)
        pltpu.async_copy(outv.at[b], out.at[pl.ds(obase, CHUNK)], osems[b])
      return 0

    lax.fori_loop(0, NCHUNK // NB, pair_body, 0)
    wait_out(0)
    wait_out(1)

  return k(user_embedding, item_embedding, u_idx, v_idx)


def _pack_table(t):
  n = t.shape[0]
  return lax.bitcast_convert_type(
      t.astype(jnp.bfloat16).reshape(n, D // 2, 2), jnp.float32)


def kernel(pos_edges, neg_edges, user_embedding, item_embedding):
  pe = pos_edges.astype(jnp.int32)
  ne = neg_edges.astype(jnp.int32)
  u_idx = jnp.concatenate([pe[0], ne[0]])
  v_idx = jnp.concatenate([pe[1], ne[1]])
  upk = _pack_table(user_embedding)
  ipk = _pack_table(item_embedding)
  scores = _edge_scores(upk, ipk, u_idx, v_idx)
  return (scores[:E, None], scores[E:, None])


# speedup vs baseline: 1.7869x; 1.7869x over previous
import functools

import jax
import jax.numpy as jnp
from jax import lax
from jax.experimental import pallas as pl
from jax.experimental.pallas import tpu as pltpu
from jax.experimental.pallas import tpu_sc as plsc

N_USERS = 10000
N_ITEMS = 10000
D = 128
E = 320000

NC = 2
NS = 16
L = 16
NW = NC * NS
N_TOTAL = 2 * E
EW = N_TOTAL // NW
CHUNK = 80
NCHUNK = EW // CHUNK
G = CHUNK // L
NB = 2


def _edge_scores(user_embedding, item_embedding, u_idx, v_idx):
  mesh = plsc.VectorSubcoreMesh(core_axis_name="c", subcore_axis_name="s")

  @functools.partial(
      pl.kernel,
      mesh=mesh,
      compiler_params=pltpu.CompilerParams(needs_layout_passes=False,
                                           use_tc_tiling_on_sc=False),
      out_type=jax.ShapeDtypeStruct((N_TOTAL,), jnp.float32),
      scratch_types=[
          pltpu.VMEM((EW,), jnp.int32),
          pltpu.VMEM((EW,), jnp.int32),
          pltpu.VMEM((NB, CHUNK, D // 2), jnp.float32),
          pltpu.VMEM((NB, CHUNK, D // 2), jnp.float32),
          pltpu.VMEM((NB, CHUNK), jnp.float32),
          pltpu.VMEM((L * L,), jnp.float32),
          pltpu.SemaphoreType.DMA,
          pltpu.SemaphoreType.DMA,
          pltpu.SemaphoreType.DMA,
          pltpu.SemaphoreType.DMA,
      ],
  )
  def k(uemb, iemb, uidx, vidx, out, uix_v, vix_v, urows, vrows, outv, mat,
        g0, g1, o0, o1):
    gsems = (g0, g1)
    osems = (o0, o1)
    wid = lax.axis_index("s") * NC + lax.axis_index("c")
    base = wid * EW
    lane = lax.iota(jnp.int32, L)

    pltpu.sync_copy(uidx.at[pl.ds(base, EW)], uix_v)
    pltpu.sync_copy(vidx.at[pl.ds(base, EW)], vix_v)

    def fire(cc, b):
      off = pl.multiple_of(cc * CHUNK, CHUNK)
      pltpu.async_copy(uemb.at[uix_v.at[pl.ds(off, CHUNK)]], urows.at[b],
                       gsems[b])
      pltpu.async_copy(iemb.at[vix_v.at[pl.ds(off, CHUNK)]], vrows.at[b],
                       gsems[b])

    def wait_gather(b):
      pltpu.make_async_copy(uemb.at[uix_v.at[pl.ds(0, CHUNK)]], urows.at[b],
                            gsems[b]).wait()
      pltpu.make_async_copy(iemb.at[vix_v.at[pl.ds(0, CHUNK)]], vrows.at[b],
                            gsems[b]).wait()

    def wait_out(b):
      pltpu.make_async_copy(outv.at[b], out.at[pl.ds(0, CHUNK)],
                            osems[b]).wait()

    fire(0, 0)

    def pair_body(i, _):
      for b in range(NB):
        cc = i * NB + b
        nb = (b + 1) % NB

        @pl.when(cc + 1 < NCHUNK)
        def _():
          fire(cc + 1, nb)

        @pl.when(cc >= NB)
        def _():
          wait_out(b)

        wait_gather(b)

        def group_body(g, _):
          accs = []
          for t in range(L):
            e = g * L + t
            acc0 = acc1 = None
            for j in range(D // (2 * L)):
              uw = plsc.bitcast(urows[b, e, pl.ds(j * L, L)], jnp.bfloat16)
              vw = plsc.bitcast(vrows[b, e, pl.ds(j * L, L)], jnp.bfloat16)
              pw = plsc.bitcast(uw * vw, jnp.uint32)
              hi = plsc.bitcast(pw & jnp.uint32(0xFFFF0000), jnp.float32)
              lo = plsc.bitcast(pw << jnp.uint32(16), jnp.float32)
              acc0 = hi if acc0 is None else acc0 + hi
              acc1 = lo if acc1 is None else acc1 + lo
            accs.append(acc0 + acc1)
          cur = accs
          for k in (1, 2, 4, 8):
            m = (lane & k) == 0
            perm = lane ^ k
            cur = [jnp.where(m, a + a[perm], b + b[perm])
                   for a, b in zip(cur[0::2], cur[1::2])]
          outv[b, pl.ds(g * L, L)] = cur[0]
          return 0

        lax.fori_loop(0, G, group_body, 0, unroll=True)
        obase = pl.multiple_of(base + cc * CHUNK, CHUNK)
        pltpu.async_copy(outv.at[b], out.at[pl.ds(obase, CHUNK)], osems[b])
      return 0

    lax.fori_loop(0, NCHUNK // NB, pair_body, 0)
    wait_out(0)
    wait_out(1)

  return k(user_embedding, item_embedding, u_idx, v_idx)


def _pack_table(t):
  n = t.shape[0]
  return lax.bitcast_convert_type(
      t.astype(jnp.bfloat16).reshape(n, D // 2, 2), jnp.float32)


def kernel(pos_edges, neg_edges, user_embedding, item_embedding):
  pe = pos_edges.astype(jnp.int32)
  ne = neg_edges.astype(jnp.int32)
  u_idx = jnp.concatenate([pe[0], ne[0]])
  v_idx = jnp.concatenate([pe[1], ne[1]])
  upk = _pack_table(user_embedding)
  ipk = _pack_table(item_embedding)
  scores = _edge_scores(upk, ipk, u_idx, v_idx)
  return (scores[:E, None], scores[E:, None])
